# baseline (speedup 1.0000x reference)
import functools

import jax
import jax.numpy as jnp
from jax.experimental import pallas as pl
from jax.experimental.pallas import tpu as pltpu

_INPUT_DIM = 768
_NUM_EXPERTS = 8
_TOPK = 2
_LOAD_BALANCING_COEF = 0.1

_BLK = 2048


def _route_block(x, wt):
    logits = jax.lax.dot_general(
        x, wt, (((1,), (0,)), ((), ())),
        preferred_element_type=jnp.float32)
    m = jnp.max(logits, axis=1, keepdims=True)
    e = jnp.exp(logits - m)
    s = jnp.sum(e, axis=1, keepdims=True)
    probs = e / s

    col = jax.lax.broadcasted_iota(jnp.int32, probs.shape, 1)
    m1 = jnp.max(probs, axis=1, keepdims=True)
    i1 = jnp.min(jnp.where(probs == m1, col, _NUM_EXPERTS), axis=1)
    oh1 = col == i1[:, None]
    pm = jnp.where(oh1, -1.0, probs)
    m2 = jnp.max(pm, axis=1, keepdims=True)
    i2 = jnp.min(jnp.where(pm == m2, col, _NUM_EXPERTS), axis=1)
    oh2 = col == i2[:, None]

    agg = jnp.sum(probs, axis=0)
    cnt = jnp.sum(oh1.astype(jnp.float32) + oh2.astype(jnp.float32), axis=0)
    return m1[:, 0], m2[:, 0], i1, i2, agg, cnt


def _router_body(xa_ref, xb_ref, wt_ref,
                 p1a_ref, p2a_ref, i1a_ref, i2a_ref,
                 p1b_ref, p2b_ref, i1b_ref, i2b_ref,
                 agg_ref, cnt_ref, loss_ref):
    step = pl.program_id(0)
    nsteps = pl.num_programs(0)

    @pl.when(step == 0)
    def _init():
        agg_ref[...] = jnp.zeros_like(agg_ref)
        cnt_ref[...] = jnp.zeros_like(cnt_ref)

    wt = wt_ref[...]
    p1a, p2a, i1a, i2a, agg_a, cnt_a = _route_block(xa_ref[...], wt)
    p1a_ref[...] = p1a
    p2a_ref[...] = p2a
    i1a_ref[...] = i1a
    i2a_ref[...] = i2a

    p1b, p2b, i1b, i2b, agg_b, cnt_b = _route_block(xb_ref[...], wt)
    p1b_ref[...] = p1b
    p2b_ref[...] = p2b
    i1b_ref[...] = i1b
    i2b_ref[...] = i2b

    agg_ref[...] += (agg_a + agg_b)[None, :]
    cnt_ref[...] += (cnt_a + cnt_b)[None, :]

    @pl.when(step == nsteps - 1)
    def _final():
        num_tokens = nsteps * _BLK * 2
        scale = (_NUM_EXPERTS * _LOAD_BALANCING_COEF
                 / (num_tokens * num_tokens * _TOPK))
        loss_ref[...] = (jnp.sum(agg_ref[...] * cnt_ref[...])
                         * scale).reshape(1, 1)


@functools.partial(jax.jit, static_argnames=("interpret",))
def _router(x, wt, interpret=False):
    n = x.shape[0]
    half = n // 2
    nblk = half // _BLK
    grid = (nblk,)
    out_shapes = (
        jax.ShapeDtypeStruct((half,), jnp.float32),
        jax.ShapeDtypeStruct((half,), jnp.float32),
        jax.ShapeDtypeStruct((half,), jnp.int32),
        jax.ShapeDtypeStruct((half,), jnp.int32),
        jax.ShapeDtypeStruct((half,), jnp.float32),
        jax.ShapeDtypeStruct((half,), jnp.float32),
        jax.ShapeDtypeStruct((half,), jnp.int32),
        jax.ShapeDtypeStruct((half,), jnp.int32),
        jax.ShapeDtypeStruct((1, _NUM_EXPERTS), jnp.float32),
        jax.ShapeDtypeStruct((1, _NUM_EXPERTS), jnp.float32),
        jax.ShapeDtypeStruct((1, 1), jnp.float32),
    )
    vec_spec = pl.BlockSpec((_BLK,), lambda i: (i,))
    acc_spec = pl.BlockSpec((1, _NUM_EXPERTS), lambda i: (0, 0))
    return pl.pallas_call(
        _router_body,
        grid=grid,
        in_specs=[
            pl.BlockSpec((_BLK, _INPUT_DIM), lambda i: (i, 0)),
            pl.BlockSpec((_BLK, _INPUT_DIM), lambda i, nblk=nblk: (i + nblk, 0)),
            pl.BlockSpec((_INPUT_DIM, _NUM_EXPERTS), lambda i: (0, 0)),
        ],
        out_specs=(
            vec_spec, vec_spec, vec_spec, vec_spec,
            vec_spec, vec_spec, vec_spec, vec_spec,
            acc_spec, acc_spec,
            pl.BlockSpec((1, 1), lambda i: (0, 0)),
        ),
        out_shape=out_shapes,
        compiler_params=pltpu.CompilerParams(
            dimension_semantics=("arbitrary",)),
        interpret=interpret,
    )(x, x, wt)


def kernel(input, W):
    x = input.reshape(-1, _INPUT_DIM)
    (p1a, p2a, i1a, i2a, p1b, p2b, i1b, i2b,
     _agg, _cnt, loss) = _router(x, W.T)
    top_probs = jnp.stack([jnp.concatenate([p1a, p1b]),
                           jnp.concatenate([p2a, p2b])], axis=1)
    top_indices = jnp.stack([jnp.concatenate([i1a, i1b]),
                             jnp.concatenate([i2a, i2b])], axis=1)
    return top_probs, top_indices, loss[0, 0]


# speedup vs baseline: 1.0307x; 1.0307x over previous
import functools

import jax
import jax.numpy as jnp
from jax.experimental import pallas as pl
from jax.experimental.pallas import tpu as pltpu

_INPUT_DIM = 768
_NUM_EXPERTS = 8
_TOPK = 2
_LOAD_BALANCING_COEF = 0.1

_BLK = 4096


def _router_body(x_ref, wt_ref, p1_ref, p2_ref, i1_ref, i2_ref,
                 agg_ref, cnt_ref, loss_ref):
    step = pl.program_id(0)
    nsteps = pl.num_programs(0)

    @pl.when(step == 0)
    def _init():
        agg_ref[...] = jnp.zeros_like(agg_ref)
        cnt_ref[...] = jnp.zeros_like(cnt_ref)

    x = x_ref[...]
    wt = wt_ref[...]
    logits = jax.lax.dot_general(
        x, wt, (((1,), (0,)), ((), ())),
        preferred_element_type=jnp.float32)

    m = jnp.max(logits, axis=1, keepdims=True)
    e = jnp.exp(logits - m)
    s = jnp.sum(e, axis=1, keepdims=True)
    probs = e / s

    col = jax.lax.broadcasted_iota(jnp.int32, probs.shape, 1)
    m1 = jnp.max(probs, axis=1, keepdims=True)
    i1 = jnp.min(jnp.where(probs == m1, col, _NUM_EXPERTS), axis=1)
    oh1 = col == i1[:, None]
    pm = jnp.where(oh1, -1.0, probs)
    m2 = jnp.max(pm, axis=1, keepdims=True)
    i2 = jnp.min(jnp.where(pm == m2, col, _NUM_EXPERTS), axis=1)
    oh2 = col == i2[:, None]

    p1_ref[...] = m1[:, 0]
    p2_ref[...] = m2[:, 0]
    i1_ref[...] = i1
    i2_ref[...] = i2

    ones = jnp.ones((1, x.shape[0]), dtype=jnp.float32)
    sel = oh1.astype(jnp.float32) + oh2.astype(jnp.float32)
    agg_ref[...] += jax.lax.dot_general(
        ones, probs, (((1,), (0,)), ((), ())),
        preferred_element_type=jnp.float32)
    cnt_ref[...] += jax.lax.dot_general(
        ones, sel, (((1,), (0,)), ((), ())),
        preferred_element_type=jnp.float32)

    @pl.when(step == nsteps - 1)
    def _final():
        num_tokens = nsteps * _BLK
        scale = (_NUM_EXPERTS * _LOAD_BALANCING_COEF
                 / (num_tokens * num_tokens * _TOPK))
        loss_ref[...] = (jnp.sum(agg_ref[...] * cnt_ref[...])
                         * scale).reshape(1, 1)


@functools.partial(jax.jit, static_argnames=("interpret",))
def _router(x, wt, interpret=False):
    n = x.shape[0]
    grid = (n // _BLK,)
    out_shapes = (
        jax.ShapeDtypeStruct((n,), jnp.float32),
        jax.ShapeDtypeStruct((n,), jnp.float32),
        jax.ShapeDtypeStruct((n,), jnp.int32),
        jax.ShapeDtypeStruct((n,), jnp.int32),
        jax.ShapeDtypeStruct((1, _NUM_EXPERTS), jnp.float32),
        jax.ShapeDtypeStruct((1, _NUM_EXPERTS), jnp.float32),
        jax.ShapeDtypeStruct((1, 1), jnp.float32),
    )
    vec_spec = pl.BlockSpec((_BLK,), lambda i: (i,))
    acc_spec = pl.BlockSpec((1, _NUM_EXPERTS), lambda i: (0, 0))
    return pl.pallas_call(
        _router_body,
        grid=grid,
        in_specs=[
            pl.BlockSpec((_BLK, _INPUT_DIM), lambda i: (i, 0)),
            pl.BlockSpec((_INPUT_DIM, _NUM_EXPERTS), lambda i: (0, 0)),
        ],
        out_specs=(
            vec_spec, vec_spec, vec_spec, vec_spec,
            acc_spec, acc_spec,
            pl.BlockSpec((1, 1), lambda i: (0, 0)),
        ),
        out_shape=out_shapes,
        compiler_params=pltpu.CompilerParams(
            dimension_semantics=("arbitrary",)),
        interpret=interpret,
    )(x, wt)


def kernel(input, W):
    x = input.reshape(-1, _INPUT_DIM)
    p1, p2, i1, i2, _agg, _cnt, loss = _router(x, W.T)
    top_probs = jnp.stack([p1, p2], axis=1)
    top_indices = jnp.stack([i1, i2], axis=1)
    return top_probs, top_indices, loss[0, 0]


# speedup vs baseline: 1.6688x; 1.6191x over previous
import functools

import jax
import jax.numpy as jnp
from jax import lax
from jax.experimental import pallas as pl
from jax.experimental.pallas import tpu as pltpu
from jax.experimental.pallas import tpu_sc as plsc

_INPUT_DIM = 768
_NUM_EXPERTS = 8
_TOPK = 2
_LOAD_BALANCING_COEF = 0.1

_N_TOKENS = 32768
_BLK = 4096
_NC, _NS, _LANES = 2, 16, 16
_NTILES = _NC * _NS
_TPT = _N_TOKENS // _NTILES
_GROUPS = _TPT // _LANES


def _logits_body(x_ref, w_ref, out_ref):
    out_ref[...] = lax.dot_general(
        w_ref[...], x_ref[...], (((1,), (1,)), ((), ())),
        preferred_element_type=jnp.float32)


def _logits_t(x, w):
    grid = (_N_TOKENS // _BLK,)
    return pl.pallas_call(
        _logits_body,
        grid=grid,
        in_specs=[
            pl.BlockSpec((_BLK, _INPUT_DIM), lambda i: (i, 0)),
            pl.BlockSpec((_NUM_EXPERTS, _INPUT_DIM), lambda i: (0, 0)),
        ],
        out_specs=pl.BlockSpec((_NUM_EXPERTS, _BLK), lambda i: (0, i)),
        out_shape=jax.ShapeDtypeStruct((_NUM_EXPERTS, _N_TOKENS), jnp.float32),
        compiler_params=pltpu.CompilerParams(
            dimension_semantics=("arbitrary",)),
    )(x, w)


def _route_body(lg_hbm, p1_hbm, p2_hbm, i1_hbm, i2_hbm, aggp_hbm, cntp_hbm,
                lg_v, p1_v, p2_v, i1_v, i2_v, agg_v, cnt_v):
    wid = lax.axis_index("s") * _NC + lax.axis_index("c")
    base = wid * _TPT
    pltpu.sync_copy(lg_hbm.at[:, pl.ds(base, _TPT)], lg_v)

    zero_f = jnp.zeros((_LANES,), jnp.float32)
    one_f = jnp.ones((_LANES,), jnp.float32)

    def group(g, carry):
        aggs, cnts = carry
        off = g * _LANES
        ls = [lg_v[e, pl.ds(off, _LANES)] for e in range(_NUM_EXPERTS)]
        m = ls[0]
        for e in range(1, _NUM_EXPERTS):
            m = jnp.maximum(m, ls[e])
        es = [jnp.exp(l - m) for l in ls]
        s = es[0]
        for e in range(1, _NUM_EXPERTS):
            s = s + es[e]
        r = one_f / s
        ps = [e_ * r for e_ in es]

        m1 = ps[0]
        i1 = jnp.zeros((_LANES,), jnp.int32)
        m2 = jnp.full((_LANES,), -1.0, jnp.float32)
        i2 = jnp.zeros((_LANES,), jnp.int32)
        for e in range(1, _NUM_EXPERTS):
            p = ps[e]
            ei = jnp.full((_LANES,), e, jnp.int32)
            gt1 = p > m1
            gt2 = p > m2
            i2 = jnp.where(gt1, i1, jnp.where(gt2, ei, i2))
            m2 = jnp.where(gt1, m1, jnp.where(gt2, p, m2))
            i1 = jnp.where(gt1, ei, i1)
            m1 = jnp.where(gt1, p, m1)

        p1_v[pl.ds(off, _LANES)] = m1
        p2_v[pl.ds(off, _LANES)] = m2
        i1_v[pl.ds(off, _LANES)] = i1
        i2_v[pl.ds(off, _LANES)] = i2

        new_aggs = tuple(a + p for a, p in zip(aggs, ps))
        new_cnts = tuple(
            c + jnp.where(i1 == e, one_f, zero_f)
            + jnp.where(i2 == e, one_f, zero_f)
            for e, c in enumerate(cnts))
        return new_aggs, new_cnts

    init = (tuple(zero_f for _ in range(_NUM_EXPERTS)),
            tuple(zero_f for _ in range(_NUM_EXPERTS)))
    aggs, cnts = lax.fori_loop(0, _GROUPS, group, init)

    for e in range(_NUM_EXPERTS):
        agg_v[pl.ds(e * _LANES, _LANES)] = aggs[e]
        cnt_v[pl.ds(e * _LANES, _LANES)] = cnts[e]

    pltpu.sync_copy(p1_v, p1_hbm.at[pl.ds(base, _TPT)])
    pltpu.sync_copy(p2_v, p2_hbm.at[pl.ds(base, _TPT)])
    pltpu.sync_copy(i1_v, i1_hbm.at[pl.ds(base, _TPT)])
    pltpu.sync_copy(i2_v, i2_hbm.at[pl.ds(base, _TPT)])
    pltpu.sync_copy(agg_v, aggp_hbm.at[wid])
    pltpu.sync_copy(cnt_v, cntp_hbm.at[wid])


_route_sc = pl.kernel(
    _route_body,
    out_type=(
        jax.ShapeDtypeStruct((_N_TOKENS,), jnp.float32),
        jax.ShapeDtypeStruct((_N_TOKENS,), jnp.float32),
        jax.ShapeDtypeStruct((_N_TOKENS,), jnp.int32),
        jax.ShapeDtypeStruct((_N_TOKENS,), jnp.int32),
        jax.ShapeDtypeStruct((_NTILES, _NUM_EXPERTS * _LANES), jnp.float32),
        jax.ShapeDtypeStruct((_NTILES, _NUM_EXPERTS * _LANES), jnp.float32),
    ),
    mesh=plsc.VectorSubcoreMesh(core_axis_name="c", subcore_axis_name="s"),
    scratch_types=[
        pltpu.VMEM((_NUM_EXPERTS, _TPT), jnp.float32),
        pltpu.VMEM((_TPT,), jnp.float32),
        pltpu.VMEM((_TPT,), jnp.float32),
        pltpu.VMEM((_TPT,), jnp.int32),
        pltpu.VMEM((_TPT,), jnp.int32),
        pltpu.VMEM((_NUM_EXPERTS * _LANES,), jnp.float32),
        pltpu.VMEM((_NUM_EXPERTS * _LANES,), jnp.float32),
    ],
)


def _loss_body(aggp_ref, cntp_ref, loss_ref):
    agg = jnp.sum(aggp_ref[...], axis=0, keepdims=True)
    cnt = jnp.sum(cntp_ref[...], axis=0, keepdims=True)
    n = _NUM_EXPERTS * _LANES
    row = lax.broadcasted_iota(jnp.int32, (n, n), 0) // _LANES
    col = lax.broadcasted_iota(jnp.int32, (n, n), 1) // _LANES
    seg = (row == col).astype(jnp.float32)
    segcnt = lax.dot_general(cnt, seg, (((1,), (0,)), ((), ())),
                             preferred_element_type=jnp.float32)
    scale = (_NUM_EXPERTS * _LOAD_BALANCING_COEF
             / (_N_TOKENS * _N_TOKENS * _TOPK))
    loss_ref[...] = (jnp.sum(agg * segcnt) * scale).reshape(1, 1)


def _loss(aggp, cntp):
    n = _NUM_EXPERTS * _LANES
    spec = pl.BlockSpec((_NTILES, n), lambda: (0, 0))
    return pl.pallas_call(
        _loss_body,
        in_specs=[spec, spec],
        out_specs=pl.BlockSpec((1, 1), lambda: (0, 0)),
        out_shape=jax.ShapeDtypeStruct((1, 1), jnp.float32),
    )(aggp, cntp)


@jax.jit
def _router(x, w):
    lg = _logits_t(x, w)
    p1, p2, i1, i2, aggp, cntp = _route_sc(lg)
    loss = _loss(aggp, cntp)
    top_probs = jnp.stack([p1, p2], axis=1)
    top_indices = jnp.stack([i1, i2], axis=1)
    return top_probs, top_indices, loss[0, 0]


def kernel(input, W):
    x = input.reshape(-1, _INPUT_DIM)
    return _router(x, W)
